# Initial kernel scaffold; baseline (speedup 1.0000x reference)
import jax
import jax.numpy as jnp
from jax.experimental import pallas as pl


def kernel(x, edge_index, W, b, gamma, beta, Wfc, bfc):
    raise NotImplementedError("write your pallas kernel here")


# speedup vs baseline: 6.7448x; 6.7448x over previous
import functools

import jax
import jax.numpy as jnp
from jax import lax
from jax.experimental import pallas as pl
from jax.experimental.pallas import tpu as pltpu
from jax.experimental.pallas import tpu_sc as plsc

N = 10000
E = 320000
D = 128
H = 128
C = 64
L = 4
EPS = 1e-5

NC = 2
NS = 16
NW = NC * NS

K = 128
CH = -(-E // (NW * K))
EPT = CH * K
E_PAD = NW * EPT

RPT = 632
N_PAD = NS * RPT

_sc_mesh = plsc.VectorSubcoreMesh(core_axis_name="c", subcore_axis_name="s")


@functools.partial(
    pl.kernel,
    out_type=jax.ShapeDtypeStruct((NC, N_PAD, H), jnp.float32),
    mesh=_sc_mesh,
    scratch_types=[
        pltpu.VMEM((CH, K), jnp.int32),
        pltpu.VMEM((K, H), jnp.float32),
        pltpu.VMEM_SHARED((N_PAD, H), jnp.float32),
    ],
)
def _deg_kernel(idx_hbm, out_hbm, idx_v, rows_v, acc):
    c = lax.axis_index("c")
    s = lax.axis_index("s")
    wid = s * NC + c

    @pl.loop(0, K)
    def _(j):
        for l in range(H // 16):
            rows_v[j, pl.ds(l * 16, 16)] = jnp.zeros((16,), jnp.float32)

    base = s * RPT
    for r in range(RPT // K):
        pltpu.sync_copy(rows_v, acc.at[pl.ds(base + r * K, K)])
    rem = RPT % K
    if rem:
        pltpu.sync_copy(rows_v.at[pl.ds(0, rem)],
                        acc.at[pl.ds(base + (RPT // K) * K, rem)])

    @pl.loop(0, K)
    def _(j):
        for l in range(H // 16):
            rows_v[j, pl.ds(l * 16, 16)] = jnp.full((16,), 1.0, jnp.float32)

    plsc.subcore_barrier()
    pltpu.sync_copy(idx_hbm.at[wid], idx_v)

    @pl.loop(0, CH)
    def _(ch):
        pltpu.sync_copy(rows_v, acc.at[idx_v.at[ch]], add=True)

    plsc.subcore_barrier()
    pltpu.sync_copy(acc.at[pl.ds(base, RPT)], out_hbm.at[c, pl.ds(base, RPT)])


@functools.partial(
    pl.kernel,
    out_type=jax.ShapeDtypeStruct((NC, N_PAD, H), jnp.float32),
    mesh=_sc_mesh,
    scratch_types=[
        pltpu.VMEM((CH, K), jnp.int32),
        pltpu.VMEM((CH, K), jnp.int32),
        pltpu.VMEM((K, H), jnp.float32),
        pltpu.VMEM_SHARED((N_PAD, H), jnp.float32),
        pltpu.SemaphoreType.DMA,
    ],
)
def _mp_kernel(h_hbm, src_hbm, dst_hbm, out_hbm, src_v, dst_v, rows_v, acc,
               sem):
    c = lax.axis_index("c")
    s = lax.axis_index("s")
    wid = s * NC + c

    @pl.loop(0, K)
    def _(j):
        for l in range(H // 16):
            rows_v[j, pl.ds(l * 16, 16)] = jnp.zeros((16,), jnp.float32)

    base = s * RPT
    for r in range(RPT // K):
        pltpu.sync_copy(rows_v, acc.at[pl.ds(base + r * K, K)])
    rem = RPT % K
    if rem:
        pltpu.sync_copy(rows_v.at[pl.ds(0, rem)],
                        acc.at[pl.ds(base + (RPT // K) * K, rem)])
    plsc.subcore_barrier()

    pltpu.sync_copy(src_hbm.at[wid], src_v)
    pltpu.sync_copy(dst_hbm.at[wid], dst_v)

    @pl.loop(0, CH)
    def _(ch):
        pltpu.async_copy(h_hbm.at[src_v.at[ch]], rows_v, sem).wait()
        pltpu.sync_copy(rows_v, acc.at[dst_v.at[ch]], add=True)

    plsc.subcore_barrier()
    pltpu.sync_copy(acc.at[pl.ds(base, RPT)], out_hbm.at[c, pl.ds(base, RPT)])


def _prep_body(x_ref, dego_ref, degi_ref, xs_ref, nin_ref, nout_ref):
    deg_out = dego_ref[0, 0:N, 0:1] + dego_ref[1, 0:N, 0:1]
    deg_in = degi_ref[0, 0:N, 0:1] + degi_ref[1, 0:N, 0:1]
    n_out = lax.rsqrt(jnp.maximum(deg_out, 1.0))
    n_in = lax.rsqrt(jnp.maximum(deg_in, 1.0))
    nout_ref[...] = n_out
    nin_ref[...] = n_in
    xs_ref[...] = x_ref[...] * n_out


def _layer_common(mp_ref, nin_ref, w_ref, b_ref, g_ref, be_ref, hprev_ref):
    m = (mp_ref[0, 0:N, :] + mp_ref[1, 0:N, :]) * nin_ref[...]
    y = jnp.dot(m, w_ref[...], preferred_element_type=jnp.float32) + b_ref[...]
    mean = jnp.mean(y, axis=0, keepdims=True)
    d = y - mean
    var = jnp.mean(d * d, axis=0, keepdims=True)
    hn = d * lax.rsqrt(var + EPS) * g_ref[...] + be_ref[...]
    if hprev_ref is not None:
        hn = hn + hprev_ref[...]
    return jnp.maximum(hn, 0.0)


def _layer0_body(mp_ref, nin_ref, nout_ref, w_ref, b_ref, g_ref, be_ref,
                 h_ref, hs_ref):
    h = _layer_common(mp_ref, nin_ref, w_ref, b_ref, g_ref, be_ref, None)
    h_ref[...] = h
    hs_ref[...] = h * nout_ref[...]


def _layer_body(mp_ref, nin_ref, nout_ref, w_ref, b_ref, g_ref, be_ref,
                hprev_ref, h_ref, hs_ref):
    h = _layer_common(mp_ref, nin_ref, w_ref, b_ref, g_ref, be_ref, hprev_ref)
    h_ref[...] = h
    hs_ref[...] = h * nout_ref[...]


def _final_body(mp_ref, nin_ref, w_ref, b_ref, g_ref, be_ref, hprev_ref,
                wfc_ref, bfc_ref, out_ref):
    h = _layer_common(mp_ref, nin_ref, w_ref, b_ref, g_ref, be_ref, hprev_ref)
    out_ref[...] = (jnp.dot(h, wfc_ref[...], preferred_element_type=jnp.float32)
                    + bfc_ref[...])


_prep_call = pl.pallas_call(
    _prep_body,
    out_shape=(jax.ShapeDtypeStruct((N, H), jnp.float32),
               jax.ShapeDtypeStruct((N, 1), jnp.float32),
               jax.ShapeDtypeStruct((N, 1), jnp.float32)),
)

_layer0_call = pl.pallas_call(
    _layer0_body,
    out_shape=(jax.ShapeDtypeStruct((N, H), jnp.float32),
               jax.ShapeDtypeStruct((N, H), jnp.float32)),
)

_layer_call = pl.pallas_call(
    _layer_body,
    out_shape=(jax.ShapeDtypeStruct((N, H), jnp.float32),
               jax.ShapeDtypeStruct((N, H), jnp.float32)),
)

_final_call = pl.pallas_call(
    _final_body,
    out_shape=jax.ShapeDtypeStruct((N, C), jnp.float32),
)


def kernel(x, edge_index, W, b, gamma, beta, Wfc, bfc):
    src = edge_index[0]
    dst = edge_index[1]
    pad = E_PAD - E
    spread = (jnp.arange(pad, dtype=jnp.int32) % 16)
    src_deg = jnp.concatenate([src, N + spread]).reshape(NW, CH, K)
    dst_p = jnp.concatenate([dst, N + spread]).reshape(NW, CH, K)
    src_mp = jnp.concatenate([src, spread]).reshape(NW, CH, K)

    deg_o = _deg_kernel(src_deg)
    deg_i = _deg_kernel(dst_p)
    xs, n_in, n_out = _prep_call(x, deg_o, deg_i)

    b2 = b.reshape(L, 1, H)
    g2 = gamma.reshape(L, 1, H)
    be2 = beta.reshape(L, 1, H)
    bfc2 = bfc.reshape(1, C)

    h = None
    hs = xs
    for i in range(L):
        mp = _mp_kernel(hs, src_mp, dst_p)
        if i == 0:
            h, hs = _layer0_call(mp, n_in, n_out, W[i], b2[i], g2[i], be2[i])
        elif i < L - 1:
            h, hs = _layer_call(mp, n_in, n_out, W[i], b2[i], g2[i], be2[i], h)
        else:
            out = _final_call(mp, n_in, W[i], b2[i], g2[i], be2[i], h,
                              Wfc, bfc2)
    return out


# speedup vs baseline: 6.7685x; 1.0035x over previous
import functools

import jax
import jax.numpy as jnp
from jax import lax
from jax.experimental import pallas as pl
from jax.experimental.pallas import tpu as pltpu
from jax.experimental.pallas import tpu_sc as plsc

N = 10000
E = 320000
D = 128
H = 128
C = 64
L = 4
EPS = 1e-5

NC = 2
NS = 16
NW = NC * NS

K = 128
CH = -(-E // (NW * K))
EPT = CH * K
E_PAD = NW * EPT

RPT = 632
N_PAD = NS * RPT

_sc_mesh = plsc.VectorSubcoreMesh(core_axis_name="c", subcore_axis_name="s")


@functools.partial(
    pl.kernel,
    out_type=jax.ShapeDtypeStruct((NC, N_PAD, H), jnp.float32),
    mesh=_sc_mesh,
    scratch_types=[
        pltpu.VMEM((CH + 2, K), jnp.int32),
        pltpu.VMEM((K, H), jnp.float32),
        pltpu.VMEM_SHARED((N_PAD, H), jnp.float32),
    ],
)
def _deg_kernel(idx_hbm, out_hbm, idx_v, rows_v, acc):
    c = lax.axis_index("c")
    s = lax.axis_index("s")
    wid = s * NC + c

    @pl.loop(0, K)
    def _(j):
        for l in range(H // 16):
            rows_v[j, pl.ds(l * 16, 16)] = jnp.zeros((16,), jnp.float32)

    base = s * RPT
    for r in range(RPT // K):
        pltpu.sync_copy(rows_v, acc.at[pl.ds(base + r * K, K)])
    rem = RPT % K
    if rem:
        pltpu.sync_copy(rows_v.at[pl.ds(0, rem)],
                        acc.at[pl.ds(base + (RPT // K) * K, rem)])

    @pl.loop(0, K)
    def _(j):
        for l in range(H // 16):
            rows_v[j, pl.ds(l * 16, 16)] = jnp.full((16,), 1.0, jnp.float32)

    plsc.subcore_barrier()
    pltpu.sync_copy(idx_hbm.at[wid], idx_v.at[pl.ds(0, CH)])

    @pl.loop(0, CH)
    def _(ch):
        pltpu.sync_copy(rows_v, acc.at[idx_v.at[ch]], add=True)

    plsc.subcore_barrier()
    pltpu.sync_copy(acc.at[pl.ds(base, RPT)], out_hbm.at[c, pl.ds(base, RPT)])


K2 = 64
CH2 = EPT // K2


@functools.partial(
    pl.kernel,
    out_type=jax.ShapeDtypeStruct((NC, N_PAD, H), jnp.float32),
    mesh=_sc_mesh,
    scratch_types=[
        pltpu.VMEM((CH2 + 2, K2), jnp.int32),
        pltpu.VMEM((2, K2), jnp.int32),
        pltpu.VMEM((2, K2), jnp.int32),
        pltpu.VMEM((2, K2, H), jnp.float32),
        pltpu.VMEM_SHARED((N_PAD, H), jnp.float32),
        pltpu.SemaphoreType.DMA,
    ],
)
def _mp_kernel(h0_hbm, h1_hbm, edge_hbm, out_hbm, pk_v, sidx_v, dstb_v, rows_v,
            acc, sem0):
    c = lax.axis_index("c")
    s = lax.axis_index("s")
    wid = s * NC + c
    hsrc = (h0_hbm, h1_hbm)

    @pl.loop(0, K2)
    def _(j):
        for l in range(H // 16):
            rows_v[0, j, pl.ds(l * 16, 16)] = jnp.zeros((16,), jnp.float32)

    base = s * RPT
    for r in range(RPT // K2):
        pltpu.sync_copy(rows_v.at[0], acc.at[pl.ds(base + r * K2, K2)])
    rem = RPT % K2
    if rem:
        pltpu.sync_copy(rows_v.at[0].at[pl.ds(0, rem)],
                        acc.at[pl.ds(base + (RPT // K2) * K2, rem)])
    plsc.subcore_barrier()

    pltpu.sync_copy(edge_hbm.at[wid], pk_v.at[pl.ds(0, CH2)])

    def _gather(ch, slot, sem):
        for l in range(K2 // 16):
            v = pk_v[ch, pl.ds(l * 16, 16)]
            sidx_v[slot, pl.ds(l * 16, 16)] = jnp.bitwise_and(v, 0xFFFF)
        pltpu.async_copy(hsrc[slot].at[sidx_v.at[slot]], rows_v.at[slot], sem)

    def _wait(slot, sem):
        pltpu.make_async_copy(hsrc[slot].at[sidx_v.at[slot]], rows_v.at[slot],
                              sem).wait()

    def _scatter(ch, slot):
        for l in range(K2 // 16):
            v = pk_v[ch, pl.ds(l * 16, 16)]
            dstb_v[slot, pl.ds(l * 16, 16)] = lax.shift_right_logical(v, 16)
        pltpu.sync_copy(rows_v.at[slot], acc.at[dstb_v.at[slot]], add=True)

    _gather(0, 0, sem0)

    @pl.loop(0, CH2 - 2, step=2)
    def _(ch):
        _wait(0, sem0)
        _gather(ch + 1, 1, sem0)
        _scatter(ch, 0)
        _wait(1, sem0)
        _gather(ch + 2, 0, sem0)
        _scatter(ch + 1, 1)

    _wait(0, sem0)
    _gather(CH2 - 1, 1, sem0)
    _scatter(CH2 - 2, 0)
    _wait(1, sem0)
    _scatter(CH2 - 1, 1)

    plsc.subcore_barrier()
    pltpu.sync_copy(acc.at[pl.ds(base, RPT)], out_hbm.at[c, pl.ds(base, RPT)])


def _prep_body(x_ref, dego_ref, degi_ref, xs_ref, xs2_ref, nin_ref,
               nout_ref):
    deg_out = dego_ref[0, 0:N, 0:1] + dego_ref[1, 0:N, 0:1]
    deg_in = degi_ref[0, 0:N, 0:1] + degi_ref[1, 0:N, 0:1]
    n_out = lax.rsqrt(jnp.maximum(deg_out, 1.0))
    n_in = lax.rsqrt(jnp.maximum(deg_in, 1.0))
    nout_ref[...] = n_out
    nin_ref[...] = n_in
    xs = x_ref[...] * n_out
    xs_ref[...] = xs
    xs2_ref[...] = xs


def _layer_common(mp_ref, nin_ref, w_ref, b_ref, g_ref, be_ref, hprev_ref):
    m = (mp_ref[0, 0:N, :] + mp_ref[1, 0:N, :]) * nin_ref[...]
    y = jnp.dot(m, w_ref[...], preferred_element_type=jnp.float32) + b_ref[...]
    mean = jnp.mean(y, axis=0, keepdims=True)
    d = y - mean
    var = jnp.mean(d * d, axis=0, keepdims=True)
    hn = d * lax.rsqrt(var + EPS) * g_ref[...] + be_ref[...]
    if hprev_ref is not None:
        hn = hn + hprev_ref[...]
    return jnp.maximum(hn, 0.0)


def _layer0_body(mp_ref, nin_ref, nout_ref, w_ref, b_ref, g_ref, be_ref,
                 h_ref, hs_ref, hs2_ref):
    h = _layer_common(mp_ref, nin_ref, w_ref, b_ref, g_ref, be_ref, None)
    h_ref[...] = h
    hs = h * nout_ref[...]
    hs_ref[...] = hs
    hs2_ref[...] = hs


def _layer_body(mp_ref, nin_ref, nout_ref, w_ref, b_ref, g_ref, be_ref,
                hprev_ref, h_ref, hs_ref, hs2_ref):
    h = _layer_common(mp_ref, nin_ref, w_ref, b_ref, g_ref, be_ref, hprev_ref)
    h_ref[...] = h
    hs = h * nout_ref[...]
    hs_ref[...] = hs
    hs2_ref[...] = hs


def _final_body(mp_ref, nin_ref, w_ref, b_ref, g_ref, be_ref, hprev_ref,
                wfc_ref, bfc_ref, out_ref):
    h = _layer_common(mp_ref, nin_ref, w_ref, b_ref, g_ref, be_ref, hprev_ref)
    out_ref[...] = (jnp.dot(h, wfc_ref[...], preferred_element_type=jnp.float32)
                    + bfc_ref[...])


_prep_call = pl.pallas_call(
    _prep_body,
    out_shape=(jax.ShapeDtypeStruct((N, H), jnp.float32),
               jax.ShapeDtypeStruct((N, H), jnp.float32),
               jax.ShapeDtypeStruct((N, 1), jnp.float32),
               jax.ShapeDtypeStruct((N, 1), jnp.float32)),
)

_layer0_call = pl.pallas_call(
    _layer0_body,
    out_shape=(jax.ShapeDtypeStruct((N, H), jnp.float32),
               jax.ShapeDtypeStruct((N, H), jnp.float32),
               jax.ShapeDtypeStruct((N, H), jnp.float32)),
)

_layer_call = pl.pallas_call(
    _layer_body,
    out_shape=(jax.ShapeDtypeStruct((N, H), jnp.float32),
               jax.ShapeDtypeStruct((N, H), jnp.float32),
               jax.ShapeDtypeStruct((N, H), jnp.float32)),
)

_final_call = pl.pallas_call(
    _final_body,
    out_shape=jax.ShapeDtypeStruct((N, C), jnp.float32),
)


def kernel(x, edge_index, W, b, gamma, beta, Wfc, bfc):
    src = edge_index[0]
    dst = edge_index[1]
    pad = E_PAD - E
    spread = (jnp.arange(pad, dtype=jnp.int32) % 16)
    src_deg = jnp.concatenate([src, N + spread]).reshape(NW, CH, K)
    dst_pad = jnp.concatenate([dst, N + spread])
    dst_p = dst_pad.reshape(NW, CH, K)
    src_pad = jnp.concatenate([src, spread])
    edges_mp = jnp.bitwise_or(src_pad,
                              jnp.left_shift(dst_pad, 16)).reshape(NW, CH2, K2)

    deg_o = _deg_kernel(src_deg)
    deg_i = _deg_kernel(dst_p)
    xs, xs2, n_in, n_out = _prep_call(x, deg_o, deg_i)

    b2 = b.reshape(L, 1, H)
    g2 = gamma.reshape(L, 1, H)
    be2 = beta.reshape(L, 1, H)
    bfc2 = bfc.reshape(1, C)

    h = None
    hs, hs2 = xs, xs2
    for i in range(L):
        mp = _mp_kernel(hs, hs2, edges_mp)
        if i == 0:
            h, hs, hs2 = _layer0_call(mp, n_in, n_out, W[i], b2[i], g2[i],
                                      be2[i])
        elif i < L - 1:
            h, hs, hs2 = _layer_call(mp, n_in, n_out, W[i], b2[i], g2[i],
                                     be2[i], h)
        else:
            out = _final_call(mp, n_in, W[i], b2[i], g2[i], be2[i], h,
                              Wfc, bfc2)
    return out


# speedup vs baseline: 6.9533x; 1.0273x over previous
import functools

import jax
import jax.numpy as jnp
from jax import lax
from jax.experimental import pallas as pl
from jax.experimental.pallas import tpu as pltpu
from jax.experimental.pallas import tpu_sc as plsc

N = 10000
E = 320000
D = 128
H = 128
C = 64
L = 4
EPS = 1e-5

NC = 2
NS = 16
NW = NC * NS

K = 128
CH = -(-E // (NW * K))
EPT = CH * K
E_PAD = NW * EPT

RPT = 632
N_PAD = NS * RPT

_sc_mesh = plsc.VectorSubcoreMesh(core_axis_name="c", subcore_axis_name="s")


@functools.partial(
    pl.kernel,
    out_type=jax.ShapeDtypeStruct((NC, N_PAD, H), jnp.float32),
    mesh=_sc_mesh,
    scratch_types=[
        pltpu.VMEM((CH + 2, K), jnp.int32),
        pltpu.VMEM((K, H), jnp.float32),
        pltpu.VMEM_SHARED((N_PAD, H), jnp.float32),
    ],
)
def _deg_kernel(idx_hbm, out_hbm, idx_v, rows_v, acc):
    c = lax.axis_index("c")
    s = lax.axis_index("s")
    wid = s * NC + c

    @pl.loop(0, K)
    def _(j):
        for l in range(H // 16):
            rows_v[j, pl.ds(l * 16, 16)] = jnp.zeros((16,), jnp.float32)

    base = s * RPT
    for r in range(RPT // K):
        pltpu.sync_copy(rows_v, acc.at[pl.ds(base + r * K, K)])
    rem = RPT % K
    if rem:
        pltpu.sync_copy(rows_v.at[pl.ds(0, rem)],
                        acc.at[pl.ds(base + (RPT // K) * K, rem)])

    @pl.loop(0, K)
    def _(j):
        for l in range(H // 16):
            rows_v[j, pl.ds(l * 16, 16)] = jnp.full((16,), 1.0, jnp.float32)

    plsc.subcore_barrier()
    pltpu.sync_copy(idx_hbm.at[wid], idx_v.at[pl.ds(0, CH)])

    @pl.loop(0, CH)
    def _(ch):
        pltpu.sync_copy(rows_v, acc.at[idx_v.at[ch]], add=True)

    plsc.subcore_barrier()
    pltpu.sync_copy(acc.at[pl.ds(base, RPT)], out_hbm.at[c, pl.ds(base, RPT)])


K3 = 256
CH3 = 40
EPT3 = CH3 * K3
E_PAD3 = NW * EPT3


@functools.partial(
    pl.kernel,
    out_type=jax.ShapeDtypeStruct((NC, N_PAD, H), jnp.float32),
    mesh=_sc_mesh,
    scratch_types=[
        pltpu.VMEM((CH3 + 2, K3), jnp.int32),
        pltpu.VMEM((K3,), jnp.int32),
        pltpu.VMEM((K3,), jnp.int32),
        pltpu.VMEM((K3, H), jnp.float32),
        pltpu.VMEM_SHARED((N_PAD, H), jnp.float32),
        pltpu.SemaphoreType.DMA,
    ],
)
def _mp_kernel(h_hbm, edge_hbm, out_hbm, pk_v, sidx_v, dstb_v, rows_v,
               acc, sem0):
    c = lax.axis_index("c")
    s = lax.axis_index("s")
    wid = s * NC + c

    @pl.loop(0, K3)
    def _(j):
        for l in range(H // 16):
            rows_v[j, pl.ds(l * 16, 16)] = jnp.zeros((16,), jnp.float32)

    base = s * RPT
    for r in range(RPT // K3):
        pltpu.sync_copy(rows_v, acc.at[pl.ds(base + r * K3, K3)])
    rem = RPT % K3
    if rem:
        pltpu.sync_copy(rows_v.at[pl.ds(0, rem)],
                        acc.at[pl.ds(base + (RPT // K3) * K3, rem)])
    plsc.subcore_barrier()

    pltpu.sync_copy(edge_hbm.at[wid], pk_v.at[pl.ds(0, CH3)])

    @pl.loop(0, CH3)
    def _(ch):
        for l in range(K3 // 16):
            v = pk_v[ch, pl.ds(l * 16, 16)]
            sidx_v[pl.ds(l * 16, 16)] = jnp.bitwise_and(v, 0xFFFF)
            dstb_v[pl.ds(l * 16, 16)] = lax.shift_right_logical(v, 16)
        pltpu.async_copy(h_hbm.at[sidx_v], rows_v, sem0).wait()
        pltpu.sync_copy(rows_v, acc.at[dstb_v], add=True)

    plsc.subcore_barrier()
    pltpu.sync_copy(acc.at[pl.ds(base, RPT)], out_hbm.at[c, pl.ds(base, RPT)])


def _prep_body(x_ref, dego_ref, degi_ref, xs_ref, nin_ref, nout_ref):
    deg_out = dego_ref[0, 0:N, 0:1] + dego_ref[1, 0:N, 0:1]
    deg_in = degi_ref[0, 0:N, 0:1] + degi_ref[1, 0:N, 0:1]
    n_out = lax.rsqrt(jnp.maximum(deg_out, 1.0))
    n_in = lax.rsqrt(jnp.maximum(deg_in, 1.0))
    nout_ref[...] = n_out
    nin_ref[...] = n_in
    xs_ref[...] = x_ref[...] * n_out


def _layer_common(mp_ref, nin_ref, w_ref, b_ref, g_ref, be_ref, hprev_ref):
    m = (mp_ref[0, 0:N, :] + mp_ref[1, 0:N, :]) * nin_ref[...]
    y = jnp.dot(m, w_ref[...], preferred_element_type=jnp.float32) + b_ref[...]
    mean = jnp.mean(y, axis=0, keepdims=True)
    d = y - mean
    var = jnp.mean(d * d, axis=0, keepdims=True)
    hn = d * lax.rsqrt(var + EPS) * g_ref[...] + be_ref[...]
    if hprev_ref is not None:
        hn = hn + hprev_ref[...]
    return jnp.maximum(hn, 0.0)


def _layer0_body(mp_ref, nin_ref, nout_ref, w_ref, b_ref, g_ref, be_ref,
                 h_ref, hs_ref):
    h = _layer_common(mp_ref, nin_ref, w_ref, b_ref, g_ref, be_ref, None)
    h_ref[...] = h
    hs_ref[...] = h * nout_ref[...]


def _layer_body(mp_ref, nin_ref, nout_ref, w_ref, b_ref, g_ref, be_ref,
                hprev_ref, h_ref, hs_ref):
    h = _layer_common(mp_ref, nin_ref, w_ref, b_ref, g_ref, be_ref, hprev_ref)
    h_ref[...] = h
    hs_ref[...] = h * nout_ref[...]


def _final_body(mp_ref, nin_ref, w_ref, b_ref, g_ref, be_ref, hprev_ref,
                wfc_ref, bfc_ref, out_ref):
    h = _layer_common(mp_ref, nin_ref, w_ref, b_ref, g_ref, be_ref, hprev_ref)
    out_ref[...] = (jnp.dot(h, wfc_ref[...], preferred_element_type=jnp.float32)
                    + bfc_ref[...])


_prep_call = pl.pallas_call(
    _prep_body,
    out_shape=(jax.ShapeDtypeStruct((N, H), jnp.float32),
               jax.ShapeDtypeStruct((N, 1), jnp.float32),
               jax.ShapeDtypeStruct((N, 1), jnp.float32)),
)

_layer0_call = pl.pallas_call(
    _layer0_body,
    out_shape=(jax.ShapeDtypeStruct((N, H), jnp.float32),
               jax.ShapeDtypeStruct((N, H), jnp.float32)),
)

_layer_call = pl.pallas_call(
    _layer_body,
    out_shape=(jax.ShapeDtypeStruct((N, H), jnp.float32),
               jax.ShapeDtypeStruct((N, H), jnp.float32)),
)

_final_call = pl.pallas_call(
    _final_body,
    out_shape=jax.ShapeDtypeStruct((N, C), jnp.float32),
)


def kernel(x, edge_index, W, b, gamma, beta, Wfc, bfc):
    src = edge_index[0]
    dst = edge_index[1]
    pad = E_PAD - E
    spread = (jnp.arange(pad, dtype=jnp.int32) % 16)
    src_deg = jnp.concatenate([src, N + spread]).reshape(NW, CH, K)
    dst_pad = jnp.concatenate([dst, N + spread])
    dst_p = dst_pad.reshape(NW, CH, K)
    pad3 = E_PAD3 - E
    spread3 = (jnp.arange(pad3, dtype=jnp.int32) % 16)
    src_pad3 = jnp.concatenate([src, spread3])
    dst_pad3 = jnp.concatenate([dst, N + spread3])
    edges_mp = jnp.bitwise_or(src_pad3,
                              jnp.left_shift(dst_pad3, 16)).reshape(
                                  NW, CH3, K3)

    deg_o = _deg_kernel(src_deg)
    deg_i = _deg_kernel(dst_p)
    xs, n_in, n_out = _prep_call(x, deg_o, deg_i)

    b2 = b.reshape(L, 1, H)
    g2 = gamma.reshape(L, 1, H)
    be2 = beta.reshape(L, 1, H)
    bfc2 = bfc.reshape(1, C)

    h = None
    hs = xs
    for i in range(L):
        mp = _mp_kernel(hs, edges_mp)
        if i == 0:
            h, hs = _layer0_call(mp, n_in, n_out, W[i], b2[i], g2[i], be2[i])
        elif i < L - 1:
            h, hs = _layer_call(mp, n_in, n_out, W[i], b2[i], g2[i], be2[i], h)
        else:
            out = _final_call(mp, n_in, W[i], b2[i], g2[i], be2[i], h,
                              Wfc, bfc2)
    return out
